# baseline (speedup 1.0000x reference)
import jax
import jax.numpy as jnp
from jax.experimental import pallas as pl

_HD = 384
_FC = 8
_CS = 8
_FS = 2
_SC2 = 2 * _HD // _FC
_SC = _SC2 // 2

_HP = jax.lax.Precision.HIGHEST


def _fold48(sq):
    pad = jnp.concatenate(
        [sq, jnp.zeros(sq.shape[:-1] + (16,), jnp.float32)], axis=-1)
    for s in (32, 16, 8, 4, 2, 1):
        pad = pad[..., :s] + pad[..., s:2 * s]
    return pad


def _cluster_kernel(xcm_ref, wp_ref, bp_ref, wm_ref, bm_ref, a_ref, b_ref,
                    out_ref):
    c2 = 2 * _HD
    side = 32
    npix = side * side
    ns = 256

    Xcm = xcm_ref[0]
    P = jax.lax.dot_general(Xcm, wp_ref[...], (((0,), (1,)), ((), ())),
                            preferred_element_type=jnp.float32)
    P = P + bp_ref[...]

    Pg = P.reshape(side // 2, 2, side, c2)
    v = Pg[:, 0] + Pg[:, 1]
    u = v.reshape(side // 2, side // 2, 2, c2)
    ct = u[:, :, 0] + u[:, :, 1]
    C = (ct * 0.25).reshape(ns, c2)

    alpha = a_ref[0, 0]
    beta = b_ref[0, 0]
    neg = jnp.float32(-jnp.inf)

    p_io = jax.lax.broadcasted_iota(jnp.int32, (npix, 1), 0)
    jq_io = jax.lax.broadcasted_iota(jnp.int32, (1, ns), 1)
    valid = ((p_io // 512 == jq_io // 128) &
             ((p_io % 32) // 16 == (jq_io % 16) // 8))
    j_iota = jax.lax.broadcasted_iota(jnp.int32, (npix, ns), 1)
    ones = jnp.ones((npix, 1), jnp.float32)

    c_iota = jax.lax.broadcasted_iota(jnp.int32, (c2, _FC), 0)
    h_iota = jax.lax.broadcasted_iota(jnp.int32, (c2, _FC), 1)
    sel = jnp.where((c_iota // _SC2 == h_iota) & (c_iota % _SC2 < _SC),
                    1.0, 0.0)
    xnorm2 = jax.lax.dot_general(P * P, sel, (((1,), (0,)), ((), ())),
                                 preferred_element_type=jnp.float32,
                                 precision=_HP)

    disp_heads = []
    for h in range(_FC):
        base = h * _SC2
        xp_pt = P[:, base:base + _SC]
        xp_val = P[:, base + _SC:base + _SC2]
        c_pt = C[:, base:base + _SC]
        c_val = C[:, base + _SC:base + _SC2]

        xn = xp_pt / jnp.maximum(jnp.sqrt(xnorm2[:, h:h + 1]), 1e-12)
        cn = c_pt / jnp.maximum(jnp.sqrt(_fold48(c_pt * c_pt)), 1e-12)
        cat_x = jnp.concatenate([xp_val, ones], axis=1)

        sim = jax.lax.dot_general(xn, cn, (((1,), (1,)), ((), ())),
                                  preferred_element_type=jnp.float32)
        Am = jnp.where(valid, alpha * sim + beta, neg)

        m = jnp.max(Am, axis=1, keepdims=True)
        idx = jnp.min(jnp.where(Am == m, j_iota, ns), axis=1, keepdims=True)
        vals = jax.nn.sigmoid(m)
        wv = jnp.where(j_iota == idx, vals, 0.0)

        numer = jax.lax.dot_general(
            wv, cat_x, (((0,), (0,)), ((), ())),
            preferred_element_type=jnp.float32, precision=_HP)
        aggn = (c_val + numer[:, :_SC]) / (1.0 + numer[:, _SC:_SC + 1])
        disp_heads.append(jax.lax.dot_general(
            wv, aggn, (((1,), (0,)), ((), ())),
            preferred_element_type=jnp.float32, precision=_HP))

    D = jnp.concatenate(disp_heads, axis=1)
    out = jax.lax.dot_general(wm_ref[...], D, (((1,), (1,)), ((), ())),
                              preferred_element_type=jnp.float32)
    out_ref[0] = out + bm_ref[...]


@jax.jit
def kernel(x, W_proj, b_proj, W_merge, b_merge, alpha, beta):
    n, c, h, w = x.shape
    npix = h * w
    c2 = W_proj.shape[0]
    c_out = W_merge.shape[0]

    xcm = x.reshape(n, c, npix)

    out_cm = pl.pallas_call(
        _cluster_kernel,
        grid=(n,),
        in_specs=[
            pl.BlockSpec((1, c, npix), lambda i: (i, 0, 0)),
            pl.BlockSpec((c2, c), lambda i: (0, 0)),
            pl.BlockSpec((1, c2), lambda i: (0, 0)),
            pl.BlockSpec((c_out, _HD), lambda i: (0, 0)),
            pl.BlockSpec((c_out, 1), lambda i: (0, 0)),
            pl.BlockSpec((1, 1), lambda i: (0, 0)),
            pl.BlockSpec((1, 1), lambda i: (0, 0)),
        ],
        out_specs=pl.BlockSpec((1, c_out, npix), lambda i: (i, 0, 0)),
        out_shape=jax.ShapeDtypeStruct((n, c_out, npix), jnp.float32),
    )(xcm, W_proj, b_proj.reshape(1, c2), W_merge, b_merge.reshape(c_out, 1),
      alpha.reshape(1, 1), beta.reshape(1, 1))

    return out_cm.reshape(n, c_out, h, w)


# speedup vs baseline: 1.0972x; 1.0972x over previous
import jax
import jax.numpy as jnp
from jax.experimental import pallas as pl

_HD = 384
_FC = 8
_CS = 8
_FS = 2
_SC2 = 2 * _HD // _FC
_SC = _SC2 // 2
_Q = _FS * _FS

_HP = jax.lax.Precision.HIGHEST


def _fold48(sq):
    pad = jnp.concatenate(
        [sq, jnp.zeros(sq.shape[:-1] + (16,), jnp.float32)], axis=-1)
    for s in (32, 16, 8, 4, 2, 1):
        pad = pad[..., :s] + pad[..., s:2 * s]
    return pad


def _cluster_kernel(xcm_ref, wp_ref, bp_ref, wm_ref, bm_ref, a_ref, b_ref,
                    out_ref):
    c2 = 2 * _HD
    side = 32
    rows = side * side
    npix = 256
    s = _CS * _CS
    half = 16

    Xcm = xcm_ref[0]
    P = jax.lax.dot_general(Xcm, wp_ref[...], (((0,), (1,)), ((), ())),
                            preferred_element_type=jnp.float32)
    P = P + bp_ref[...]

    P = P.reshape(_FS, half, _FS, half, c2).transpose(0, 2, 1, 3, 4)
    P = P.reshape(rows, c2)

    Pg = P.reshape(_Q, half // 2, 2, half, c2)
    v = Pg[:, :, 0] + Pg[:, :, 1]
    u = v.reshape(_Q, half // 2, half // 2, 2, c2)
    ct = u[:, :, :, 0] + u[:, :, :, 1]
    C = (ct * 0.25).reshape(_Q * s, c2)

    alpha = a_ref[0, 0]
    beta = b_ref[0, 0]
    j_iota = jax.lax.broadcasted_iota(jnp.int32, (npix, s), 1)
    ones = jnp.ones((rows, 1), jnp.float32)

    c_iota = jax.lax.broadcasted_iota(jnp.int32, (c2, _FC), 0)
    h_iota = jax.lax.broadcasted_iota(jnp.int32, (c2, _FC), 1)
    sel = jnp.where((c_iota // _SC2 == h_iota) & (c_iota % _SC2 < _SC),
                    1.0, 0.0)
    xnorm2 = jax.lax.dot_general(P * P, sel, (((1,), (0,)), ((), ())),
                                 preferred_element_type=jnp.float32,
                                 precision=_HP)

    disp_heads = []
    for h in range(_FC):
        base = h * _SC2
        xp_pt = P[:, base:base + _SC]
        xp_val = P[:, base + _SC:base + _SC2]
        c_pt = C[:, base:base + _SC]
        c_val = C[:, base + _SC:base + _SC2]

        xn = xp_pt / jnp.maximum(jnp.sqrt(xnorm2[:, h:h + 1]), 1e-12)
        cn = c_pt / jnp.maximum(jnp.sqrt(_fold48(c_pt * c_pt)), 1e-12)
        cat_x = jnp.concatenate([xp_val, ones], axis=1)

        disp_q = []
        for qd in range(_Q):
            xn_q = xn[qd * npix:(qd + 1) * npix]
            cn_q = cn[qd * s:(qd + 1) * s]
            sim = jax.lax.dot_general(xn_q, cn_q, (((1,), (1,)), ((), ())),
                                      preferred_element_type=jnp.float32)
            A = alpha * sim + beta

            m = jnp.max(A, axis=1, keepdims=True)
            idx = jnp.min(jnp.where(A == m, j_iota, s), axis=1,
                          keepdims=True)
            vals = jax.nn.sigmoid(m)
            wv = jnp.where(j_iota == idx, vals, 0.0)

            numer = jax.lax.dot_general(
                wv, cat_x[qd * npix:(qd + 1) * npix],
                (((0,), (0,)), ((), ())),
                preferred_element_type=jnp.float32, precision=_HP)
            aggn = (c_val[qd * s:(qd + 1) * s] + numer[:, :_SC]) / (
                1.0 + numer[:, _SC:_SC + 1])
            disp_q.append(jax.lax.dot_general(
                wv, aggn, (((1,), (0,)), ((), ())),
                preferred_element_type=jnp.float32, precision=_HP))
        disp_heads.append(jnp.concatenate(disp_q, axis=0))

    D = jnp.concatenate(disp_heads, axis=1)
    D = D.reshape(_FS, _FS, half, half, _HD).transpose(0, 2, 1, 3, 4)
    D = D.reshape(rows, _HD)
    out = jax.lax.dot_general(wm_ref[...], D, (((1,), (1,)), ((), ())),
                              preferred_element_type=jnp.float32)
    out_ref[0] = out + bm_ref[...]


@jax.jit
def kernel(x, W_proj, b_proj, W_merge, b_merge, alpha, beta):
    n, c, h, w = x.shape
    npix = h * w
    c2 = W_proj.shape[0]
    c_out = W_merge.shape[0]

    xcm = x.reshape(n, c, npix)

    out_cm = pl.pallas_call(
        _cluster_kernel,
        grid=(n,),
        in_specs=[
            pl.BlockSpec((1, c, npix), lambda i: (i, 0, 0)),
            pl.BlockSpec((c2, c), lambda i: (0, 0)),
            pl.BlockSpec((1, c2), lambda i: (0, 0)),
            pl.BlockSpec((c_out, _HD), lambda i: (0, 0)),
            pl.BlockSpec((c_out, 1), lambda i: (0, 0)),
            pl.BlockSpec((1, 1), lambda i: (0, 0)),
            pl.BlockSpec((1, 1), lambda i: (0, 0)),
        ],
        out_specs=pl.BlockSpec((1, c_out, npix), lambda i: (i, 0, 0)),
        out_shape=jax.ShapeDtypeStruct((n, c_out, npix), jnp.float32),
    )(xcm, W_proj, b_proj.reshape(1, c2), W_merge, b_merge.reshape(c_out, 1),
      alpha.reshape(1, 1), beta.reshape(1, 1))

    return out_cm.reshape(n, c_out, h, w)
